# baseline (speedup 1.0000x reference)
import jax
import jax.numpy as jnp
from jax import lax
from jax.experimental import pallas as pl
from jax.experimental.pallas import tpu as pltpu
from jax.experimental.pallas import tpu_sc as plsc

B, F_IN, F_TGT, C, R = 4096, 480, 512, 8, 4
D = C * R
L = 16
NW = 32
B_PER_W = B // NW
IN_W = F_IN * D
OUT_W = F_TGT * D
ZROW = IN_W
GRP = OUT_W // L
UNROLL = 8


def _pack_body(ll_hbm, perm_hbm, out_hbm,
               perm_v, srct_v, in0, in1, rows0, rows1,
               semi0, semi1, semo0, semo1):
    wid = lax.axis_index("s") * 2 + lax.axis_index("c")
    base_b = wid * B_PER_W

    pltpu.sync_copy(perm_hbm, perm_v)

    zf = jnp.zeros((L,), jnp.float32)
    for buf in (in0, in1):
        buf[pl.ds(ZROW, L)] = zf
        buf[pl.ds(ZROW + L, L)] = zf

    iota = jnp.arange(L, dtype=jnp.int32)

    def build(tj, carry):
        v = perm_v[pl.ds(tj * L, L)]
        sv = jnp.where(v < F_IN, v, F_IN) * D
        jidx = (iota + tj * L) * D
        for w in range(D):
            plsc.store_scatter(srct_v, [jidx + w], sv + w)
        return carry

    lax.fori_loop(0, F_TGT // L, build, 0)

    def in_slice(b):
        return ll_hbm.at[pl.ds(b * IN_W, IN_W)]

    def out_slice(b):
        return out_hbm.at[pl.ds(b * OUT_W, OUT_W)]

    def permute(inb, rowsb):
        def pg(g, carry):
            o = g * (L * UNROLL)
            for u in range(UNROLL):
                sidx = srct_v[pl.ds(o + u * L, L)]
                rowsb[pl.ds(o + u * L, L)] = plsc.load_gather(inb, [sidx])
            return carry

        lax.fori_loop(0, GRP // UNROLL, pg, 0)

    pltpu.async_copy(in_slice(base_b), in0.at[pl.ds(0, IN_W)], semi0)
    pltpu.async_copy(in_slice(base_b + 1), in1.at[pl.ds(0, IN_W)], semi1)

    def bloop(i2, carry):
        for p, inb, rowsb, semi, semo in (
                (0, in0, rows0, semi0, semo0),
                (1, in1, rows1, semi1, semo1)):
            b = base_b + 2 * i2 + p
            pltpu.make_async_copy(
                in_slice(b), inb.at[pl.ds(0, IN_W)], semi).wait()

            @pl.when(i2 >= 1)
            def _():
                pltpu.make_async_copy(rowsb, out_slice(b - 2), semo).wait()

            permute(inb, rowsb)
            pltpu.async_copy(rowsb, out_slice(b), semo)

            @pl.when(i2 < (B_PER_W // 2) - 1)
            def _():
                pltpu.async_copy(
                    in_slice(b + 2), inb.at[pl.ds(0, IN_W)], semi)
        return carry

    lax.fori_loop(0, B_PER_W // 2, bloop, 0)

    last = base_b + B_PER_W - 2
    pltpu.make_async_copy(rows0, out_slice(last), semo0).wait()
    pltpu.make_async_copy(rows1, out_slice(last + 1), semo1).wait()


def kernel(ll, perm):
    ll_flat = ll.reshape(B * IN_W)
    mesh = plsc.VectorSubcoreMesh(core_axis_name="c", subcore_axis_name="s")
    out = pl.kernel(
        _pack_body,
        mesh=mesh,
        compiler_params=pltpu.CompilerParams(
            use_tc_tiling_on_sc=False, needs_layout_passes=False),
        out_type=jax.ShapeDtypeStruct((B * OUT_W,), jnp.float32),
        scratch_types=[
            pltpu.VMEM((F_TGT,), jnp.int32),
            pltpu.VMEM((OUT_W,), jnp.int32),
            pltpu.VMEM((IN_W + D,), jnp.float32),
            pltpu.VMEM((IN_W + D,), jnp.float32),
            pltpu.VMEM((OUT_W,), jnp.float32),
            pltpu.VMEM((OUT_W,), jnp.float32),
            pltpu.SemaphoreType.DMA,
            pltpu.SemaphoreType.DMA,
            pltpu.SemaphoreType.DMA,
            pltpu.SemaphoreType.DMA,
        ],
    )(ll_flat, perm)
    return out.reshape(B, F_TGT, C, R)


# speedup vs baseline: 17.2512x; 17.2512x over previous
import jax
import jax.numpy as jnp
from jax import lax
from jax.experimental import pallas as pl
from jax.experimental.pallas import tpu as pltpu
from jax.experimental.pallas import tpu_sc as plsc

B, F_IN, F_TGT, C, R = 4096, 480, 512, 8, 4
M = C * R
L = 16
TL = 128
BH = B // TL
FH = F_TGT // TL
BT = 64
ST = BT + 1
N_BT = B // BT
BSUB = 16
N_SUB = BT // BSUB
JG = F_TGT // L


def _pack_body(ll_hbm, perm_hbm, out_hbm,
               perm_v, srct_v, tile0, tile1, outb0, outb1,
               semt0, semt1, semo0, semo1):
    m = lax.axis_index("s") * 2 + lax.axis_index("c")
    c = m // R
    r = m % R

    pltpu.sync_copy(perm_hbm, perm_v)

    for t in range(JG):
        v = perm_v[pl.ds(t * L, L)]
        srct_v[pl.ds(t * L, L)] = jnp.where(v < F_IN, v, F_IN)

    zf = jnp.zeros((L,), jnp.float32)
    for buf in (tile0, tile1):
        for q in range(BT // L):
            buf[F_IN, pl.ds(q * L, L)] = zf

    def stage(u, tile, semt):
        return pltpu.make_async_copy(
            ll_hbm.at[pl.ds(0, F_IN), c, u // 2, r,
                      pl.ds((u % 2) * BT, BT)],
            tile.at[pl.ds(0, F_IN), pl.ds(0, BT)], semt)

    def flush(gp, outb, semo):
        b0 = gp * BSUB
        return pltpu.make_async_copy(
            outb,
            out_hbm.at[pl.ds(b0, BSUB), c, pl.ds(0, FH), r, pl.ds(0, TL)],
            semo)

    def do_tile(u, tile, semt, tile_next, semt_next):
        stage(u, tile, semt).wait()

        @pl.when(u < N_BT - 1)
        def _():
            stage(u + 1, tile_next, semt_next).start()

        for s in range(N_SUB):
            outb = (outb0, outb1)[s % 2]
            semo = (semo0, semo1)[s % 2]
            gp = u * N_SUB + s

            @pl.when(gp >= 2)
            def _():
                flush(gp - 2, outb, semo).wait()

            def bloop(bi, bcarry):
                col = jnp.broadcast_to(
                    (s * BSUB + bi).astype(jnp.int32), (L,))
                for g in range(JG):
                    ridx = srct_v[pl.ds(g * L, L)]
                    outb[bi, g // 8, pl.ds((g % 8) * L, L)] = (
                        plsc.load_gather(tile, [ridx, col]))
                return bcarry

            lax.fori_loop(0, BSUB, bloop, 0)
            flush(gp, outb, semo).start()

    stage(0, tile0, semt0).start()

    def uloop(u2, carry):
        do_tile(2 * u2, tile0, semt0, tile1, semt1)
        do_tile(2 * u2 + 1, tile1, semt1, tile0, semt0)
        return carry

    lax.fori_loop(0, N_BT // 2, uloop, 0)

    flush(N_BT * N_SUB - 2, outb0, semo0).wait()
    flush(N_BT * N_SUB - 1, outb1, semo1).wait()


def kernel(ll, perm):
    llv = (ll.transpose(1, 2, 3, 0)
             .reshape(F_IN, C, R, BH, TL)
             .transpose(0, 1, 3, 2, 4))
    mesh = plsc.VectorSubcoreMesh(core_axis_name="c", subcore_axis_name="s")
    out5 = pl.kernel(
        _pack_body,
        mesh=mesh,
        compiler_params=pltpu.CompilerParams(
            use_tc_tiling_on_sc=False, needs_layout_passes=False),
        out_type=jax.ShapeDtypeStruct((B, C, FH, R, TL), jnp.float32),
        scratch_types=[
            pltpu.VMEM((F_TGT,), jnp.int32),
            pltpu.VMEM((F_TGT,), jnp.int32),
            pltpu.VMEM((F_IN + 1, ST), jnp.float32),
            pltpu.VMEM((F_IN + 1, ST), jnp.float32),
            pltpu.VMEM((BSUB, FH, TL), jnp.float32),
            pltpu.VMEM((BSUB, FH, TL), jnp.float32),
            pltpu.SemaphoreType.DMA,
            pltpu.SemaphoreType.DMA,
            pltpu.SemaphoreType.DMA,
            pltpu.SemaphoreType.DMA,
        ],
    )(llv, perm)
    return (out5.transpose(0, 2, 4, 1, 3)
                .reshape(B, F_TGT, C, R))
